# baseline (speedup 1.0000x reference)
import functools

import jax
import jax.numpy as jnp
from jax import lax
from jax.experimental import pallas as pl
from jax.experimental.pallas import tpu as pltpu
from jax.experimental.pallas import tpu_sc as plsc

_NC = 2
_NS = 16
_NW = _NC * _NS

_T_ROWS = 288
_D_ROWS = 7
_EMB = 32
_OUT_W = 2 * _EMB

_SB = 8
_BB = 128
_STRIDE = _EMB + 1
_T_REP = 2
_T_REP_OFF = _T_ROWS * _STRIDE + 8
_D_REP_OFF = _D_ROWS * _STRIDE


def _make_sc_lookup(n_s, n_b):
    assert n_s % _SB == 0 and n_b % (_NW * _BB) == 0
    b_per_w = n_b // _NW
    n_sblk = n_s // _SB
    n_bsub = b_per_w // _BB

    mesh = plsc.VectorSubcoreMesh(
        core_axis_name="c", subcore_axis_name="s",
        num_cores=_NC, num_subcores=_NS,
    )

    @functools.partial(
        pl.kernel,
        out_type=jax.ShapeDtypeStruct((n_s, _OUT_W, n_b), jnp.float32),
        mesh=mesh,
        compiler_params=pltpu.CompilerParams(needs_layout_passes=False),
        scratch_types=[
            pltpu.VMEM((_T_ROWS * _STRIDE,), jnp.float32),
            pltpu.VMEM((_D_ROWS * _STRIDE,), jnp.float32),
            pltpu.VMEM((_SB, _BB), jnp.int32),
            pltpu.VMEM((_SB, _BB), jnp.int32),
            pltpu.VMEM((_SB // 2, _OUT_W, _BB), jnp.float32),
            pltpu.VMEM((_SB // 2, _OUT_W, _BB), jnp.float32),
            pltpu.SemaphoreType.DMA,
            pltpu.SemaphoreType.DMA,
        ],
    )
    def sc_lookup(tt_hbm, dt_hbm, ttab_hbm, dtab_hbm, out_hbm,
                  ttab_v, dtab_v, it_v, id_v, buf0_v, buf1_v, sem0, sem1):
        wid = lax.axis_index("s") * _NC + lax.axis_index("c")
        b_base = wid * b_per_w
        pltpu.sync_copy(ttab_hbm, ttab_v)
        pltpu.sync_copy(dtab_hbm, dtab_v)
        hsb = _SB // 2
        lane = jnp.arange(16, dtype=jnp.int32)
        day_regs = [
            plsc.load_gather(dtab_v, [lane + (j * _STRIDE + half * 16)])
            for j in range(_D_ROWS) for half in range(2)
        ]

        def sblk(i, carry):
            s0 = i * _SB

            def bsub(j, carry2):
                b0 = b_base + j * _BB
                not_first = jnp.logical_or(i > 0, j > 0)
                pltpu.sync_copy(tt_hbm.at[pl.ds(s0, _SB), pl.ds(b0, _BB)], it_v)
                pltpu.sync_copy(dt_hbm.at[pl.ds(s0, _SB), pl.ds(b0, _BB)], id_v)

                for ph, buf_v, sem in ((0, buf0_v, sem0), (1, buf1_v, sem1)):
                    dst = out_hbm.at[pl.ds(s0 + ph * hsb, hsb),
                                     slice(None), pl.ds(b0, _BB)]

                    @pl.when(not_first)
                    def _drain(buf_v=buf_v, dst=dst, sem=sem):
                        pltpu.make_async_copy(buf_v, dst, sem).wait()

                    @plsc.parallel_loop(0, hsb)
                    def rows(r, buf_v=buf_v, ph=ph):
                        ri = r + ph * hsb
                        t33s = [it_v[ri, pl.ds(k * 16, 16)] * _STRIDE
                                for k in range(_BB // 16)]
                        d16s = [id_v[ri, pl.ds(k * 16, 16)]
                                for k in range(_BB // 16)]
                        for half in range(2):
                            dregs = [day_regs[2 * j + half] for j in range(_D_ROWS)]

                            def esweep(e16, carry3, half=half, dregs=dregs):
                                e = half * 16 + e16
                                sp = jnp.full((16,), e16, dtype=jnp.int32)
                                bcs = [dr.at[sp].get(mode="promise_in_bounds")
                                       for dr in dregs]
                                for k in range(_BB // 16):
                                    sl = pl.ds(k * 16, 16)
                                    buf_v[r, e, sl] = plsc.load_gather(
                                        ttab_v, [t33s[k] + e])
                                    d = d16s[k]
                                    bit0 = (d & 1) != 0
                                    bit1 = (d & 2) != 0
                                    low = jnp.where(
                                        bit1,
                                        jnp.where(bit0, bcs[3], bcs[2]),
                                        jnp.where(bit0, bcs[1], bcs[0]))
                                    high = jnp.where(
                                        bit1, bcs[6],
                                        jnp.where(bit0, bcs[5], bcs[4]))
                                    buf_v[r, _EMB + e, sl] = jnp.where(
                                        (d & 4) != 0, high, low)
                                return carry3

                            lax.fori_loop(0, 16, esweep, 0)

                    pltpu.async_copy(buf_v, dst, sem)
                return carry2

            lax.fori_loop(0, n_bsub, bsub, 0)
            return carry

        lax.fori_loop(0, n_sblk, sblk, 0)

        for buf_v, sem in ((buf0_v, sem0), (buf1_v, sem1)):
            pltpu.make_async_copy(
                buf_v,
                out_hbm.at[pl.ds(0, hsb), slice(None), pl.ds(b_base, _BB)],
                sem,
            ).wait()

    return sc_lookup


def kernel(time_idx, day_idx, time_table, day_table):
    b, s = time_idx.shape
    tt = time_idx.T.astype(jnp.int32)
    dt = day_idx.T.astype(jnp.int32)
    ttab = jnp.pad(time_table.astype(jnp.float32),
                   ((0, 0), (0, _STRIDE - _EMB))).reshape(_T_ROWS * _STRIDE)
    dtab = jnp.pad(day_table.astype(jnp.float32),
                   ((0, 0), (0, _STRIDE - _EMB))).reshape(_D_ROWS * _STRIDE)
    out3 = _make_sc_lookup(s, b)(tt, dt, ttab, dtab)
    return out3.transpose(2, 0, 1)


# speedup vs baseline: 1.3236x; 1.3236x over previous
import functools

import jax
import jax.numpy as jnp
from jax import lax
from jax.experimental import pallas as pl
from jax.experimental.pallas import tpu as pltpu
from jax.experimental.pallas import tpu_sc as plsc

_NC = 2
_NS = 16
_NW = _NC * _NS

_T_ROWS = 288
_D_ROWS = 7
_EMB = 32
_OUT_W = 2 * _EMB

_SB = 8
_BB = 128
_STRIDE = _EMB + 1
_T_REP = 2
_T_REP_OFF = _T_ROWS * _STRIDE + 8
_D_REP_OFF = _D_ROWS * _STRIDE


def _make_sc_lookup(n_s, n_b):
    assert n_s % _SB == 0 and n_b % (_NW * _BB) == 0
    b_per_w = n_b // _NW
    n_sblk = n_s // _SB
    n_bsub = b_per_w // _BB

    mesh = plsc.VectorSubcoreMesh(
        core_axis_name="c", subcore_axis_name="s",
        num_cores=_NC, num_subcores=_NS,
    )

    @functools.partial(
        pl.kernel,
        out_type=jax.ShapeDtypeStruct((n_s, _OUT_W, n_b), jnp.float32),
        mesh=mesh,
        compiler_params=pltpu.CompilerParams(needs_layout_passes=False),
        scratch_types=[
            pltpu.VMEM((_T_ROWS * _STRIDE,), jnp.float32),
            pltpu.VMEM((_D_ROWS * _STRIDE,), jnp.float32),
            pltpu.VMEM((_SB, _BB), jnp.int32),
            pltpu.VMEM((_SB, _BB), jnp.int32),
            pltpu.VMEM((_SB // 2, _OUT_W, _BB), jnp.float32),
            pltpu.VMEM((_SB // 2, _OUT_W, _BB), jnp.float32),
            pltpu.SemaphoreType.DMA,
            pltpu.SemaphoreType.DMA,
        ],
    )
    def sc_lookup(tt_hbm, dt_hbm, ttab_hbm, dtab_hbm, out_hbm,
                  ttab_v, dtab_v, it_v, id_v, buf0_v, buf1_v, sem0, sem1):
        wid = lax.axis_index("s") * _NC + lax.axis_index("c")
        b_base = wid * b_per_w
        pltpu.sync_copy(ttab_hbm, ttab_v)
        pltpu.sync_copy(dtab_hbm, dtab_v)
        hsb = _SB // 2

        def sblk(i, carry):
            s0 = i * _SB

            def bsub(j, carry2):
                b0 = b_base + j * _BB
                not_first = jnp.logical_or(i > 0, j > 0)
                pltpu.sync_copy(tt_hbm.at[pl.ds(s0, _SB), pl.ds(b0, _BB)], it_v)
                pltpu.sync_copy(dt_hbm.at[pl.ds(s0, _SB), pl.ds(b0, _BB)], id_v)

                for ph, buf_v, sem in ((0, buf0_v, sem0), (1, buf1_v, sem1)):
                    dst = out_hbm.at[pl.ds(s0 + ph * hsb, hsb),
                                     slice(None), pl.ds(b0, _BB)]

                    @pl.when(not_first)
                    def _drain(buf_v=buf_v, dst=dst, sem=sem):
                        pltpu.make_async_copy(buf_v, dst, sem).wait()

                    @plsc.parallel_loop(0, hsb)
                    def rows(r, buf_v=buf_v, ph=ph):
                        ri = r + ph * hsb
                        for k in range(_BB // 16):
                            sl = pl.ds(k * 16, 16)
                            t16 = it_v[ri, sl] * _STRIDE
                            d16 = id_v[ri, sl] * _STRIDE
                            tv = [plsc.load_gather(ttab_v, [t16 + e])
                                  for e in range(_EMB)]
                            for e in range(_EMB):
                                buf_v[r, e, sl] = tv[e]
                            dv = [plsc.load_gather(dtab_v, [d16 + e])
                                  for e in range(_EMB)]
                            for e in range(_EMB):
                                buf_v[r, _EMB + e, sl] = dv[e]

                    pltpu.async_copy(buf_v, dst, sem)
                return carry2

            lax.fori_loop(0, n_bsub, bsub, 0)
            return carry

        lax.fori_loop(0, n_sblk, sblk, 0)

        for buf_v, sem in ((buf0_v, sem0), (buf1_v, sem1)):
            pltpu.make_async_copy(
                buf_v,
                out_hbm.at[pl.ds(0, hsb), slice(None), pl.ds(b_base, _BB)],
                sem,
            ).wait()

    return sc_lookup


def kernel(time_idx, day_idx, time_table, day_table):
    b, s = time_idx.shape
    tt = time_idx.T.astype(jnp.int32)
    dt = day_idx.T.astype(jnp.int32)
    ttab = jnp.pad(time_table.astype(jnp.float32),
                   ((0, 0), (0, _STRIDE - _EMB))).reshape(_T_ROWS * _STRIDE)
    dtab = jnp.pad(day_table.astype(jnp.float32),
                   ((0, 0), (0, _STRIDE - _EMB))).reshape(_D_ROWS * _STRIDE)
    out3 = _make_sc_lookup(s, b)(tt, dt, ttab, dtab)
    return out3.transpose(2, 0, 1)


# speedup vs baseline: 1.3284x; 1.0036x over previous
import functools

import jax
import jax.numpy as jnp
from jax import lax
from jax.experimental import pallas as pl
from jax.experimental.pallas import tpu as pltpu
from jax.experimental.pallas import tpu_sc as plsc

_NC = 2
_NS = 16
_NW = _NC * _NS

_T_ROWS = 288
_D_ROWS = 7
_EMB = 32
_OUT_W = 2 * _EMB

_SB = 8
_BB = 128
_STRIDE = _EMB + 1


def _make_sc_lookup(n_s, n_b):
    assert n_s % _SB == 0 and n_b % (_NW * _BB) == 0
    b_per_w = n_b // _NW
    n_sblk = n_s // _SB
    n_bsub = b_per_w // _BB

    mesh = plsc.VectorSubcoreMesh(
        core_axis_name="c", subcore_axis_name="s",
        num_cores=_NC, num_subcores=_NS,
    )

    @functools.partial(
        pl.kernel,
        out_type=jax.ShapeDtypeStruct((n_s, _OUT_W, n_b), jnp.float32),
        mesh=mesh,
        compiler_params=pltpu.CompilerParams(needs_layout_passes=False),
        scratch_types=[
            pltpu.VMEM((_T_ROWS * _STRIDE,), jnp.float32),
            pltpu.VMEM((_D_ROWS * _STRIDE,), jnp.float32),
            pltpu.VMEM((_SB, _BB), jnp.int32),
            pltpu.VMEM((_SB, _BB), jnp.int32),
            pltpu.VMEM((_SB // 2, _OUT_W, _BB), jnp.float32),
            pltpu.VMEM((_SB // 2, _OUT_W, _BB), jnp.float32),
            pltpu.SemaphoreType.DMA,
            pltpu.SemaphoreType.DMA,
        ],
    )
    def sc_lookup(tt_hbm, dt_hbm, ttab_hbm, dtab_hbm, out_hbm,
                  ttab_v, dtab_v, it_v, id_v, buf0_v, buf1_v, sem0, sem1):
        wid = lax.axis_index("s") * _NC + lax.axis_index("c")
        b_base = wid * b_per_w
        pltpu.sync_copy(ttab_hbm, ttab_v)
        pltpu.sync_copy(dtab_hbm, dtab_v)
        hsb = _SB // 2

        def sblk(i, carry):
            s0 = i * _SB

            def bsub(j, carry2):
                b0 = b_base + j * _BB
                not_first = jnp.logical_or(i > 0, j > 0)
                pltpu.sync_copy(tt_hbm.at[pl.ds(s0, _SB), pl.ds(b0, _BB)], it_v)
                pltpu.sync_copy(dt_hbm.at[pl.ds(s0, _SB), pl.ds(b0, _BB)], id_v)

                for ph, buf_v, sem in ((0, buf0_v, sem0), (1, buf1_v, sem1)):
                    dst = out_hbm.at[pl.ds(s0 + ph * hsb, hsb),
                                     slice(None), pl.ds(b0, _BB)]

                    @pl.when(not_first)
                    def _drain(buf_v=buf_v, dst=dst, sem=sem):
                        pltpu.make_async_copy(buf_v, dst, sem).wait()

                    @plsc.parallel_loop(0, hsb)
                    def rows(r, buf_v=buf_v, ph=ph):
                        ri = r + ph * hsb
                        for k in range(_BB // 16):
                            sl = pl.ds(k * 16, 16)
                            t16 = it_v[ri, sl] * _STRIDE
                            d16 = id_v[ri, sl] * _STRIDE
                            tv = [plsc.load_gather(ttab_v, [t16 + e])
                                  for e in range(_EMB)]
                            for e in range(_EMB):
                                buf_v[r, e, sl] = tv[e]
                            dv = [plsc.load_gather(dtab_v, [d16 + e])
                                  for e in range(_EMB)]
                            for e in range(_EMB):
                                buf_v[r, _EMB + e, sl] = dv[e]

                    pltpu.async_copy(buf_v, dst, sem)
                return carry2

            lax.fori_loop(0, n_bsub, bsub, 0)
            return carry

        lax.fori_loop(0, n_sblk, sblk, 0)

        for buf_v, sem in ((buf0_v, sem0), (buf1_v, sem1)):
            pltpu.make_async_copy(
                buf_v,
                out_hbm.at[pl.ds(0, hsb), slice(None), pl.ds(b_base, _BB)],
                sem,
            ).wait()

    return sc_lookup


def kernel(time_idx, day_idx, time_table, day_table):
    b, s = time_idx.shape
    tt = time_idx.T.astype(jnp.int32)
    dt = day_idx.T.astype(jnp.int32)
    ttab = jnp.pad(time_table.astype(jnp.float32),
                   ((0, 0), (0, _STRIDE - _EMB))).reshape(_T_ROWS * _STRIDE)
    dtab = jnp.pad(day_table.astype(jnp.float32),
                   ((0, 0), (0, _STRIDE - _EMB))).reshape(_D_ROWS * _STRIDE)
    out3 = _make_sc_lookup(s, b)(tt, dt, ttab, dtab)
    return out3.transpose(2, 0, 1)
